# Initial kernel scaffold; baseline (speedup 1.0000x reference)
import jax
import jax.numpy as jnp
from jax.experimental import pallas as pl


def kernel(x, nodes, nbr1, nbr2, A1_Wl, A1_Wr, A1_Wlt, A1_Wrt, A1_bl, A1_br, A1_blt, A1_brt, A2_Wl, A2_Wr, A2_Wlt, A2_Wrt, A2_bl, A2_br, A2_blt, A2_brt, MTG_W, MTG_b, pw1, pb1, pw2, pb2, pw3, pb3):
    raise NotImplementedError("write your pallas kernel here")


# speedup vs baseline: 3.4817x; 3.4817x over previous
import functools

import jax
import jax.numpy as jnp
from jax.experimental import pallas as pl
from jax.experimental.pallas import tpu as pltpu

B = 10000
N = 100000
D = 128
H1 = 128
H2 = 64
C = 64
T = 3
S0, S1 = 5, 2

BS = 400
G = B // BS

STEPS = [(0, 0, 0), (0, 1, 0), (1, 0, 1), (2, 0, 2), (2, 1, 1), (2, 2, 0)]


def _dense_body(h0_ref, h1_ref, n2_ref,
                w1l_ref, w1r_ref, b1_ref,
                w2l_ref, w2r_ref, b2_ref,
                pw_ref, mtg_ref, cb_ref, out_ref):
    f32 = jnp.float32
    h0 = h0_ref[...]
    acc = jnp.zeros((BS, C), f32)
    v1 = {}
    v2 = {}
    for si, (t, c, _k) in enumerate(STEPS):
        h1 = h1_ref[t]
        n1 = (h1[0] + h1[1] + h1[2] + h1[3] + h1[4]) * (1.0 / S0)
        n2 = n2_ref[t] * (1.0 / S1)
        h1f = h1.reshape(S0 * BS, D)
        n2f = n2.reshape(S0 * BS, D)
        xcat = jnp.concatenate([h0, h1f], axis=0)
        ncat = jnp.concatenate([n1, n2f], axis=0)
        u = (jnp.dot(xcat, w1l_ref[c], preferred_element_type=f32)
             + jnp.dot(ncat, w1r_ref[c], preferred_element_type=f32)
             + b1_ref[c][None, :])
        ux = u[:, :H1]
        ut = u[:, H1:]
        v = v1.get(c, 0.0) + ut
        spk = (v >= 1.0).astype(f32)
        v1[c] = v * (1.0 - spk)
        outs = jax.nn.sigmoid(ux) * spk
        h0n = outs[:BS]
        nb = outs[BS:].reshape(S0, BS, H1)
        nb = (nb[0] + nb[1] + nb[2] + nb[3] + nb[4]) * (1.0 / S0)
        u2 = (jnp.dot(h0n, w2l_ref[c], preferred_element_type=f32)
              + jnp.dot(nb, w2r_ref[c], preferred_element_type=f32)
              + b2_ref[c][None, :])
        ux2 = u2[:, :H2]
        ut2 = u2[:, H2:]
        v_ = v2.get(c, 0.0) + ut2
        spk2 = (v_ >= 1.0).astype(f32)
        v2[c] = v_ * (1.0 - spk2)
        o = jax.nn.sigmoid(ux2) * spk2
        acc = acc + o * pw_ref[si][None, :]
    out_ref[...] = jnp.dot(acc, mtg_ref[...], preferred_element_type=f32) \
        + cb_ref[...][None, :]


@functools.partial(jax.jit, static_argnums=())
def _dense(h0, h1, n2s, w1l, w1r, b1, w2l, w2r, b2, pw, mtg, cb):
    grid = (G,)
    zero = lambda i: (0,) * 3
    return pl.pallas_call(
        _dense_body,
        grid=grid,
        in_specs=[
            pl.BlockSpec((BS, D), lambda i: (i, 0)),
            pl.BlockSpec((T, S0, BS, D), lambda i: (0, 0, i, 0)),
            pl.BlockSpec((T, S0, BS, D), lambda i: (0, 0, i, 0)),
            pl.BlockSpec((T, D, 2 * H1), lambda i: (0, 0, 0)),
            pl.BlockSpec((T, D, 2 * H1), lambda i: (0, 0, 0)),
            pl.BlockSpec((T, 2 * H1), lambda i: (0, 0)),
            pl.BlockSpec((T, H1, 2 * H2), lambda i: (0, 0, 0)),
            pl.BlockSpec((T, H1, 2 * H2), lambda i: (0, 0, 0)),
            pl.BlockSpec((T, 2 * H2), lambda i: (0, 0)),
            pl.BlockSpec((len(STEPS), H2), lambda i: (0, 0)),
            pl.BlockSpec((H2, C), lambda i: (0, 0)),
            pl.BlockSpec((C,), lambda i: (0,)),
        ],
        out_specs=pl.BlockSpec((BS, C), lambda i: (i, 0)),
        out_shape=jax.ShapeDtypeStruct((B, C), jnp.float32),
        compiler_params=pltpu.CompilerParams(
            dimension_semantics=("arbitrary",),
        ),
    )(h0, h1, n2s, w1l, w1r, b1, w2l, w2r, b2, pw, mtg, cb)


def kernel(x, nodes, nbr1, nbr2,
           A1_Wl, A1_Wr, A1_Wlt, A1_Wrt, A1_bl, A1_br, A1_blt, A1_brt,
           A2_Wl, A2_Wr, A2_Wlt, A2_Wrt, A2_bl, A2_br, A2_blt, A2_brt,
           MTG_W, MTG_b, pw1, pb1, pw2, pb2, pw3, pb3):
    nodes = nodes.astype(jnp.int32)
    nbr1 = nbr1.astype(jnp.int32)
    nbr2 = nbr2.astype(jnp.int32)

    idx1 = nbr1.reshape(T, B, S0).transpose(0, 2, 1)
    idx2 = nbr2.reshape(T, B, S0, S1).transpose(0, 2, 1, 3)

    h0 = x[nodes]
    h1 = x[idx1.reshape(-1)].reshape(T, S0, B, D)
    n2s = (x[idx2[..., 0].reshape(-1)]
           + x[idx2[..., 1].reshape(-1)]).reshape(T, S0, B, D)

    w1l = jnp.concatenate([A1_Wl.transpose(0, 2, 1),
                           A1_Wlt.transpose(0, 2, 1)], axis=2)
    w1r = jnp.concatenate([A1_Wr.transpose(0, 2, 1),
                           A1_Wrt.transpose(0, 2, 1)], axis=2)
    b1 = jnp.concatenate([A1_bl + A1_br, A1_blt + A1_brt], axis=1)
    w2l = jnp.concatenate([A2_Wl.transpose(0, 2, 1),
                           A2_Wlt.transpose(0, 2, 1)], axis=2)
    w2r = jnp.concatenate([A2_Wr.transpose(0, 2, 1),
                           A2_Wrt.transpose(0, 2, 1)], axis=2)
    b2 = jnp.concatenate([A2_bl + A2_br, A2_blt + A2_brt], axis=1)

    pw = jnp.stack([pw1[:, 0], pw2[:, 0], pw1[:, 1],
                    pw1[:, 2], pw2[:, 1], pw3[:, 0]], axis=0)
    mtg = MTG_W.T * (1.0 / 3.0)
    cb = ((pb1 + pb2 + pb3) * (1.0 / 3.0)) @ MTG_W.T + MTG_b

    return _dense(h0, h1, n2s, w1l, w1r, b1, w2l, w2r, b2, pw, mtg, cb)
